# Initial kernel scaffold; baseline (speedup 1.0000x reference)
import jax
import jax.numpy as jnp
from jax.experimental import pallas as pl


def kernel(x, edge_index, edge_type, W, b):
    raise NotImplementedError("write your pallas kernel here")


# speedup vs baseline: 6.3418x; 6.3418x over previous
import jax
import jax.numpy as jnp
from jax import lax
from jax.experimental import pallas as pl
from jax.experimental.pallas import tpu as pltpu
from jax.experimental.pallas import tpu_sc as plsc

_N = 10000
_E = 320000
_R = 8
_D = 128
_RN = _R * _N
_NC = 2
_NS = 16
_NW = _NC * _NS
_EW = _E // _NW
_C = 80
_NCH = _EW // _C
_L = 16
_BN = 1000
_RB = 624
_RREM = _N - _NS * _RB


def _deg_body(dstf, etf, degp, hist, dstb, etb):
    c = lax.axis_index("c")
    s = lax.axis_index("s")
    wid = s * _NC + c
    pltpu.sync_copy(dstf.at[pl.ds(wid * _EW, _EW)], dstb)
    pltpu.sync_copy(etf.at[pl.ds(wid * _EW, _EW)], etb)
    zeros = jnp.zeros((_L,), jnp.int32)

    def _zero(i, carry):
        hist[pl.ds(i * _L, _L)] = zeros
        return carry

    lax.fori_loop(0, _RN // _L, _zero, 0)
    ones = jnp.ones((_L,), jnp.int32)

    def _edges(i, carry):
        sl = pl.ds(i * _L, _L)
        seg = etb[sl] * _N + dstb[sl]
        plsc.addupdate_scatter(hist, [seg], ones)
        return carry

    lax.fori_loop(0, _EW // _L, _edges, 0)
    pltpu.sync_copy(hist, degp.at[pl.ds(wid * _RN, _RN)])


def _agg_body(tt, winv, srcf, dstf, etf, outp,
              srcb, dstb, etb, rows, fidxb, didxb, widxb, wbuf, zbuf,
              out_sh, winv_sh, sem1, sem2):
    c = lax.axis_index("c")
    s = lax.axis_index("s")
    wid = s * _NC + c
    pltpu.sync_copy(srcf.at[pl.ds(wid * _EW, _EW)], srcb)
    pltpu.sync_copy(dstf.at[pl.ds(wid * _EW, _EW)], dstb)
    pltpu.sync_copy(etf.at[pl.ds(wid * _EW, _EW)], etb)

    @pl.when(s == 0)
    def _stage_winv():
        pltpu.sync_copy(winv, winv_sh)

    zf = jnp.zeros((_L,), jnp.float32)
    for i in range(_RREM):
        for q in range(_D // _L):
            zbuf[i, pl.ds(q * _L, _L)] = zf

    def _zo(m, carry):
        pltpu.sync_copy(zbuf, out_sh.at[pl.ds(s * _RB + m * _RREM, _RREM)])
        return carry

    lax.fori_loop(0, _RB // _RREM, _zo, 0)

    @pl.when(s == _NS - 1)
    def _zlast():
        pltpu.sync_copy(zbuf, out_sh.at[pl.ds(_NS * _RB, _RREM)])

    plsc.subcore_barrier()

    def _chunk(j, carry):
        base = j * _C
        for k in range(_C // _L):
            sl = pl.ds(k * _L, _L)
            esl = pl.ds(base + k * _L, _L)
            tv = etb[esl]
            dv = dstb[esl]
            fidxb[sl] = tv * _N + srcb[esl]
            widxb[sl] = tv * _N + dv
            didxb[sl] = dv
        cp1 = pltpu.async_copy(tt.at[fidxb], rows, sem1)
        cp2 = pltpu.async_copy(winv_sh.at[widxb], wbuf, sem2)
        cp1.wait()
        cp2.wait()
        for k in range(_C // _L):
            wvec = wbuf[pl.ds(k * _L, _L)]
            for i16 in range(_L):
                i = k * _L + i16
                wv = jnp.full((_L,), wvec[i16], jnp.float32)
                for q in range(_D // _L):
                    qs = pl.ds(q * _L, _L)
                    rows[i, qs] = rows[i, qs] * wv
        pltpu.sync_copy(rows, out_sh.at[didxb], add=True)
        return carry

    lax.fori_loop(0, _NCH, _chunk, 0)
    plsc.subcore_barrier()
    pltpu.sync_copy(out_sh.at[pl.ds(s * _RB, _RB)],
                    outp.at[c, pl.ds(s * _RB, _RB)])

    @pl.when(s == _NS - 1)
    def _clast():
        pltpu.sync_copy(out_sh.at[pl.ds(_NS * _RB, _RREM)],
                        outp.at[c, pl.ds(_NS * _RB, _RREM)])


def _mm_body(x_ref, w_ref, o_ref):
    o_ref[...] = jnp.dot(x_ref[...], w_ref[0],
                         preferred_element_type=jnp.float32)


def _winv_body(dp_ref, o_ref):
    ssum = jnp.sum(dp_ref[...], axis=0).astype(jnp.float32)
    o_ref[...] = (1.0 / jnp.maximum(ssum, 1.0))[None, :]


def _fin_body(p_ref, b_ref, o_ref):
    o_ref[...] = p_ref[0] + p_ref[1] + b_ref[...]


def kernel(x, edge_index, edge_type, W, b):
    srcf = edge_index[0]
    dstf = edge_index[1]
    etf = edge_type

    mesh = plsc.VectorSubcoreMesh(core_axis_name="c", subcore_axis_name="s")

    degp = pl.kernel(
        _deg_body,
        out_type=jax.ShapeDtypeStruct((_NW * _RN,), jnp.int32),
        mesh=mesh,
        scratch_types=[
            pltpu.VMEM((_RN,), jnp.int32),
            pltpu.VMEM((_EW,), jnp.int32),
            pltpu.VMEM((_EW,), jnp.int32),
        ],
        compiler_params=pltpu.CompilerParams(needs_layout_passes=False),
    )(dstf, etf)

    tt = pl.pallas_call(
        _mm_body,
        grid=(_R, _N // _BN),
        in_specs=[
            pl.BlockSpec((_BN, _D), lambda r, n: (n, 0)),
            pl.BlockSpec((1, _D, _D), lambda r, n: (r, 0, 0)),
        ],
        out_specs=pl.BlockSpec((_BN, _D),
                               lambda r, n: (r * (_N // _BN) + n, 0)),
        out_shape=jax.ShapeDtypeStruct((_RN, _D), jnp.float32),
    )(x, W)

    winv2 = pl.pallas_call(
        _winv_body,
        out_shape=jax.ShapeDtypeStruct((1, _RN), jnp.float32),
    )(degp.reshape(_NW, _RN))
    winv = winv2.reshape(_RN)

    outp = pl.kernel(
        _agg_body,
        out_type=jax.ShapeDtypeStruct((_NC, _N, _D), jnp.float32),
        mesh=mesh,
        scratch_types=[
            pltpu.VMEM((_EW,), jnp.int32),
            pltpu.VMEM((_EW,), jnp.int32),
            pltpu.VMEM((_EW,), jnp.int32),
            pltpu.VMEM((_C, _D), jnp.float32),
            pltpu.VMEM((_C,), jnp.int32),
            pltpu.VMEM((_C,), jnp.int32),
            pltpu.VMEM((_C,), jnp.int32),
            pltpu.VMEM((_C,), jnp.float32),
            pltpu.VMEM((_RREM, _D), jnp.float32),
            pltpu.VMEM_SHARED((_N, _D), jnp.float32),
            pltpu.VMEM_SHARED((_RN,), jnp.float32),
            pltpu.SemaphoreType.DMA,
            pltpu.SemaphoreType.DMA,
        ],
        compiler_params=pltpu.CompilerParams(needs_layout_passes=False),
    )(tt, winv, srcf, dstf, etf)

    out = pl.pallas_call(
        _fin_body,
        grid=(_N // _BN,),
        in_specs=[
            pl.BlockSpec((_NC, _BN, _D), lambda i: (0, i, 0)),
            pl.BlockSpec((1, _D), lambda i: (0, 0)),
        ],
        out_specs=pl.BlockSpec((_BN, _D), lambda i: (i, 0)),
        out_shape=jax.ShapeDtypeStruct((_N, _D), jnp.float32),
    )(outp, b.reshape(1, _D))
    return out


# speedup vs baseline: 8.5724x; 1.3517x over previous
import jax
import jax.numpy as jnp
from jax import lax
from jax.experimental import pallas as pl
from jax.experimental.pallas import tpu as pltpu
from jax.experimental.pallas import tpu_sc as plsc

_N = 10000
_E = 320000
_R = 8
_D = 128
_RN = _R * _N
_NC = 2
_NS = 16
_NW = _NC * _NS
_EW = _E // _NW
_C = 80
_NCH = _EW // _C
_L = 16
_BN = 1000
_RB = 624
_RREM = _N - _NS * _RB
_PART = 5
_EPP = _EW // _PART
_CPP = _EPP // _C


def _deg_body(dstf, etf, degp, hist, dstb, etb):
    c = lax.axis_index("c")
    s = lax.axis_index("s")
    wid = s * _NC + c
    pltpu.sync_copy(dstf.at[pl.ds(wid * _EW, _EW)], dstb)
    pltpu.sync_copy(etf.at[pl.ds(wid * _EW, _EW)], etb)
    zeros = jnp.zeros((_L,), jnp.int32)

    def _zero(i, carry):
        hist[pl.ds(i * _L, _L)] = zeros
        return carry

    lax.fori_loop(0, _RN // _L, _zero, 0)
    ones = jnp.ones((_L,), jnp.int32)

    def _edges(i, carry):
        sl = pl.ds(i * _L, _L)
        seg = etb[sl] * _N + dstb[sl]
        plsc.addupdate_scatter(hist, [seg], ones)
        return carry

    lax.fori_loop(0, _EW // _L, _edges, 0)
    pltpu.sync_copy(hist, degp.at[pl.ds(wid * _RN, _RN)])


def _agg_body(tt, winv, srcf, dstf, etf, outp,
              srcb, dstb, etb, rows0, rows1, fidxb0, fidxb1,
              didxb0, didxb1, widxb0, widxb1, wbuf0, wbuf1,
              out_sh, winv_sh, semr0, semr1, semw0, semw1):
    c = lax.axis_index("c")
    s = lax.axis_index("s")
    wid = s * _NC + c
    rows = (rows0, rows1)
    fidxb = (fidxb0, fidxb1)
    didxb = (didxb0, didxb1)
    widxb = (widxb0, widxb1)
    wbuf = (wbuf0, wbuf1)
    semr = (semr0, semr1)
    semw = (semw0, semw1)

    @pl.when(s == 0)
    def _stage_winv():
        pltpu.sync_copy(winv, winv_sh)

    zf = jnp.zeros((_L,), jnp.float32)
    for i in range(_RREM):
        for q in range(_D // _L):
            rows0[i, pl.ds(q * _L, _L)] = zf

    def _zo(m, carry):
        pltpu.sync_copy(rows0.at[pl.ds(0, _RREM)],
                        out_sh.at[pl.ds(s * _RB + m * _RREM, _RREM)])
        return carry

    lax.fori_loop(0, _RB // _RREM, _zo, 0)

    @pl.when(s == _NS - 1)
    def _zlast():
        pltpu.sync_copy(rows0.at[pl.ds(0, _RREM)],
                        out_sh.at[pl.ds(_NS * _RB, _RREM)])

    plsc.subcore_barrier()

    def _fire(cc, p):
        base = cc * _C
        for k in range(_C // _L):
            sl = pl.ds(k * _L, _L)
            esl = pl.ds(base + k * _L, _L)
            tv = etb[esl]
            dv = dstb[esl]
            fidxb[p][sl] = tv * _N + srcb[esl]
            widxb[p][sl] = tv * _N + dv
            didxb[p][sl] = dv
        pltpu.async_copy(tt.at[fidxb[p]], rows[p], semr[p])
        pltpu.async_copy(winv_sh.at[widxb[p]], wbuf[p], semw[p])

    def _process(p):
        pltpu.make_async_copy(tt.at[fidxb[p]], rows[p], semr[p]).wait()
        pltpu.make_async_copy(winv_sh.at[widxb[p]], wbuf[p],
                              semw[p]).wait()
        def _scale(k, cc):
            wvec = wbuf[p][pl.ds(k * _L, _L)]
            for i16 in range(_L):
                i = k * _L + i16
                wv = jnp.full((_L,), wvec[i16], jnp.float32)
                for q in range(_D // _L):
                    qs = pl.ds(q * _L, _L)
                    rows[p][i, qs] = rows[p][i, qs] * wv
            return cc

        lax.fori_loop(0, _C // _L, _scale, 0)
        pltpu.sync_copy(rows[p], out_sh.at[didxb[p]], add=True)

    for h in range(_PART):
        off = wid * _EW + h * _EPP
        pltpu.sync_copy(srcf.at[pl.ds(off, _EPP)], srcb)
        pltpu.sync_copy(dstf.at[pl.ds(off, _EPP)], dstb)
        pltpu.sync_copy(etf.at[pl.ds(off, _EPP)], etb)
        _fire(0, 0)

        def _pair(j, carry):
            _fire(2 * j + 1, 1)
            _process(0)
            _fire(2 * j + 2, 0)
            _process(1)
            return carry

        lax.fori_loop(0, (_CPP - 1) // 2, _pair, 0)
        _process(0)

    plsc.subcore_barrier()
    pltpu.sync_copy(out_sh.at[pl.ds(s * _RB, _RB)],
                    outp.at[c, pl.ds(s * _RB, _RB)])

    @pl.when(s == _NS - 1)
    def _clast():
        pltpu.sync_copy(out_sh.at[pl.ds(_NS * _RB, _RREM)],
                        outp.at[c, pl.ds(_NS * _RB, _RREM)])


def _mm_body(x_ref, w_ref, o_ref):
    o_ref[...] = jnp.dot(x_ref[...], w_ref[0],
                         preferred_element_type=jnp.float32)


def _winv_body(dp_ref, o_ref):
    ssum = jnp.sum(dp_ref[...], axis=0).astype(jnp.float32)
    o_ref[...] = (1.0 / jnp.maximum(ssum, 1.0))[None, :]


def _fin_body(p_ref, b_ref, o_ref):
    o_ref[...] = p_ref[0] + p_ref[1] + b_ref[...]


def kernel(x, edge_index, edge_type, W, b):
    srcf = edge_index[0]
    dstf = edge_index[1]
    etf = edge_type

    mesh = plsc.VectorSubcoreMesh(core_axis_name="c", subcore_axis_name="s")

    degp = pl.kernel(
        _deg_body,
        out_type=jax.ShapeDtypeStruct((_NW * _RN,), jnp.int32),
        mesh=mesh,
        scratch_types=[
            pltpu.VMEM((_RN,), jnp.int32),
            pltpu.VMEM((_EW,), jnp.int32),
            pltpu.VMEM((_EW,), jnp.int32),
        ],
        compiler_params=pltpu.CompilerParams(needs_layout_passes=False),
    )(dstf, etf)

    tt = pl.pallas_call(
        _mm_body,
        grid=(_R, _N // _BN),
        in_specs=[
            pl.BlockSpec((_BN, _D), lambda r, n: (n, 0)),
            pl.BlockSpec((1, _D, _D), lambda r, n: (r, 0, 0)),
        ],
        out_specs=pl.BlockSpec((_BN, _D),
                               lambda r, n: (r * (_N // _BN) + n, 0)),
        out_shape=jax.ShapeDtypeStruct((_RN, _D), jnp.float32),
    )(x, W)

    winv2 = pl.pallas_call(
        _winv_body,
        out_shape=jax.ShapeDtypeStruct((1, _RN), jnp.float32),
    )(degp.reshape(_NW, _RN))
    winv = winv2.reshape(_RN)

    outp = pl.kernel(
        _agg_body,
        out_type=jax.ShapeDtypeStruct((_NC, _N, _D), jnp.float32),
        mesh=mesh,
        scratch_types=[
            pltpu.VMEM((_EPP,), jnp.int32),
            pltpu.VMEM((_EPP,), jnp.int32),
            pltpu.VMEM((_EPP,), jnp.int32),
            pltpu.VMEM((_C, _D), jnp.float32),
            pltpu.VMEM((_C, _D), jnp.float32),
            pltpu.VMEM((_C,), jnp.int32),
            pltpu.VMEM((_C,), jnp.int32),
            pltpu.VMEM((_C,), jnp.int32),
            pltpu.VMEM((_C,), jnp.int32),
            pltpu.VMEM((_C,), jnp.int32),
            pltpu.VMEM((_C,), jnp.int32),
            pltpu.VMEM((_C,), jnp.float32),
            pltpu.VMEM((_C,), jnp.float32),
            pltpu.VMEM_SHARED((_N, _D), jnp.float32),
            pltpu.VMEM_SHARED((_RN,), jnp.float32),
            pltpu.SemaphoreType.DMA,
            pltpu.SemaphoreType.DMA,
            pltpu.SemaphoreType.DMA,
            pltpu.SemaphoreType.DMA,
        ],
        compiler_params=pltpu.CompilerParams(needs_layout_passes=False),
    )(tt, winv, srcf, dstf, etf)

    out = pl.pallas_call(
        _fin_body,
        grid=(_N // _BN,),
        in_specs=[
            pl.BlockSpec((_NC, _BN, _D), lambda i: (0, i, 0)),
            pl.BlockSpec((1, _D), lambda i: (0, 0)),
        ],
        out_specs=pl.BlockSpec((_BN, _D), lambda i: (i, 0)),
        out_shape=jax.ShapeDtypeStruct((_N, _D), jnp.float32),
    )(outp, b.reshape(1, _D))
    return out
